# baseline (speedup 1.0000x reference)
import jax
import jax.numpy as jnp
from jax.experimental import pallas as pl
from jax.experimental.pallas import tpu as pltpu

_LANE = 128
_SUBLANE = 8


def _round_up(n, m):
    return ((n + m - 1) // m) * m


def _make_mlp_kernel(n_chunks, hc):
    def _mlp_kernel(x_ref, w1_ref, b1_ref, w2_ref, b2_ref, o_ref):
        x = x_ref[...]
        y = None
        for c in range(n_chunks):
            lo = c * hc
            hi = lo + hc
            h = jnp.dot(x, w1_ref[:, lo:hi],
                        preferred_element_type=jnp.float32)
            h = jnp.maximum(h + b1_ref[:, lo:hi], 0.0)
            p = jnp.dot(h, w2_ref[lo:hi, :],
                        preferred_element_type=jnp.float32)
            y = p if y is None else y + p
        o_ref[...] = (y + b2_ref[...]).astype(o_ref.dtype)
    return _mlp_kernel


def kernel(x, w1, b1, w2, b2, *, batch_tile=1024, h_chunk=1024):
    B, D_in = x.shape
    H = w1.shape[1]
    D_out = w2.shape[1]
    dtype = x.dtype

    b1 = b1.reshape(1, H).astype(jnp.float32)
    b2 = b2.reshape(1, D_out).astype(jnp.float32)

    D_in_p = _round_up(D_in, _LANE)
    H_p = _round_up(H, _LANE)
    D_out_p = _round_up(D_out, _LANE)
    tb = min(batch_tile, _round_up(B, _SUBLANE))
    B_p = _round_up(B, tb)

    xp = jnp.pad(x, ((0, B_p - B), (0, D_in_p - D_in)))
    w1p = jnp.pad(w1, ((0, D_in_p - D_in), (0, H_p - H)))
    b1p = jnp.pad(b1, ((0, 0), (0, H_p - H)))
    w2p = jnp.pad(w2, ((0, H_p - H), (0, D_out_p - D_out)))
    b2p = jnp.pad(b2, ((0, 0), (0, D_out_p - D_out)))

    hc = min(h_chunk, H_p)
    n_chunks = -(-H_p // hc)
    if n_chunks * hc != H_p:
        hc, n_chunks = H_p, 1

    n_tiles = B_p // tb

    out_p = pl.pallas_call(
        _make_mlp_kernel(n_chunks, hc),
        out_shape=jax.ShapeDtypeStruct((B_p, D_out_p), dtype),
        grid_spec=pl.GridSpec(
            grid=(n_tiles,),
            in_specs=[
                pl.BlockSpec((tb, D_in_p), lambda i: (i, 0)),
                pl.BlockSpec((D_in_p, H_p), lambda i: (0, 0)),
                pl.BlockSpec((1, H_p), lambda i: (0, 0)),
                pl.BlockSpec((H_p, D_out_p), lambda i: (0, 0)),
                pl.BlockSpec((1, D_out_p), lambda i: (0, 0)),
            ],
            out_specs=pl.BlockSpec((tb, D_out_p), lambda i: (i, 0)),
        ),
        compiler_params=pltpu.CompilerParams(
            dimension_semantics=("arbitrary",)),
    )(xp, w1p, b1p, w2p, b2p)

    if B_p != B or D_out_p != D_out:
        out_p = out_p[:B, :D_out]
    return out_p


# speedup vs baseline: 1.0280x; 1.0280x over previous
import jax
import jax.numpy as jnp
from jax.experimental import pallas as pl
from jax.experimental.pallas import tpu as pltpu

_LANE = 128
_SUBLANE = 8


def _round_up(n, m):
    return ((n + m - 1) // m) * m


def _make_mlp_kernel(n_chunks, hc):
    def _mlp_kernel(x_ref, b1_ref, b2_ref, w1_hbm, w2_hbm, o_ref,
                    w1_vmem, w2_vmem, sem1, sem2):
        i = pl.program_id(0)

        def w1_copy(c):
            lo = c * hc
            return pltpu.make_async_copy(
                w1_hbm.at[:, pl.ds(lo, hc)],
                w1_vmem.at[:, pl.ds(lo, hc)],
                sem1.at[c])

        def w2_copy(c):
            lo = c * hc
            return pltpu.make_async_copy(
                w2_hbm.at[pl.ds(lo, hc), :],
                w2_vmem.at[pl.ds(lo, hc), :],
                sem2.at[c])

        def body(wait):
            x = x_ref[...]
            y = None
            for c in range(n_chunks):
                if wait:
                    w1_copy(c).wait()
                    w2_copy(c).wait()
                lo = c * hc
                hi = lo + hc
                h = jnp.dot(x, w1_vmem[:, lo:hi],
                            preferred_element_type=jnp.float32)
                h = jnp.maximum(h + b1_ref[:, lo:hi], 0.0)
                p = jnp.dot(h, w2_vmem[lo:hi, :],
                            preferred_element_type=jnp.float32)
                y = p if y is None else y + p
            o_ref[...] = (y + b2_ref[...]).astype(o_ref.dtype)

        @pl.when(i == 0)
        def _():
            for c in range(n_chunks):
                w1_copy(c).start()
                w2_copy(c).start()
            body(wait=True)

        @pl.when(i != 0)
        def _():
            body(wait=False)
    return _mlp_kernel


def kernel(x, w1, b1, w2, b2, *, batch_tile=512, h_chunk=1024):
    B, D_in = x.shape
    H = w1.shape[1]
    D_out = w2.shape[1]
    dtype = x.dtype

    b1 = b1.reshape(1, H).astype(jnp.float32)
    b2 = b2.reshape(1, D_out).astype(jnp.float32)

    D_in_p = _round_up(D_in, _LANE)
    H_p = _round_up(H, _LANE)
    D_out_p = _round_up(D_out, _LANE)
    tb = min(batch_tile, _round_up(B, _SUBLANE))
    B_p = _round_up(B, tb)

    xp = jnp.pad(x, ((0, B_p - B), (0, D_in_p - D_in)))
    w1p = jnp.pad(w1, ((0, D_in_p - D_in), (0, H_p - H)))
    b1p = jnp.pad(b1, ((0, 0), (0, H_p - H)))
    w2p = jnp.pad(w2, ((0, H_p - H), (0, D_out_p - D_out)))
    b2p = jnp.pad(b2, ((0, 0), (0, D_out_p - D_out)))

    hc = min(h_chunk, H_p)
    n_chunks = -(-H_p // hc)
    if n_chunks * hc != H_p:
        hc, n_chunks = H_p, 1

    n_tiles = B_p // tb

    out_p = pl.pallas_call(
        _make_mlp_kernel(n_chunks, hc),
        out_shape=jax.ShapeDtypeStruct((B_p, D_out_p), dtype),
        grid=(n_tiles,),
        in_specs=[
            pl.BlockSpec((tb, D_in_p), lambda i: (i, 0)),
            pl.BlockSpec((1, H_p), lambda i: (0, 0)),
            pl.BlockSpec((1, D_out_p), lambda i: (0, 0)),
            pl.BlockSpec(memory_space=pltpu.MemorySpace.HBM),
            pl.BlockSpec(memory_space=pltpu.MemorySpace.HBM),
        ],
        out_specs=pl.BlockSpec((tb, D_out_p), lambda i: (i, 0)),
        scratch_shapes=[
            pltpu.VMEM((D_in_p, H_p), jnp.float32),
            pltpu.VMEM((H_p, D_out_p), jnp.float32),
            pltpu.SemaphoreType.DMA((n_chunks,)),
            pltpu.SemaphoreType.DMA((n_chunks,)),
        ],
        compiler_params=pltpu.CompilerParams(
            dimension_semantics=("arbitrary",)),
    )(xp, b1p, b2p, w1p, w2p)

    if B_p != B or D_out_p != D_out:
        out_p = out_p[:B, :D_out]
    return out_p


# speedup vs baseline: 1.0325x; 1.0044x over previous
import jax
import jax.numpy as jnp
from jax.experimental import pallas as pl
from jax.experimental.pallas import tpu as pltpu

_LANE = 128
_SUBLANE = 8


def _round_up(n, m):
    return ((n + m - 1) // m) * m


def _make_mlp_kernel(hc_stream, n_stream, hc, n_chunks):
    def _mlp_kernel(x_ref, b1_ref, b2_ref, w1_hbm, w2_hbm, o_ref,
                    w1_vmem, w2_vmem, sem1, sem2):
        i = pl.program_id(0)

        def w1_copy(c):
            lo = c * hc_stream
            return pltpu.make_async_copy(
                w1_hbm.at[:, pl.ds(lo, hc_stream)],
                w1_vmem.at[:, pl.ds(lo, hc_stream)],
                sem1.at[c])

        def w2_copy(c):
            lo = c * hc_stream
            return pltpu.make_async_copy(
                w2_hbm.at[pl.ds(lo, hc_stream), :],
                w2_vmem.at[pl.ds(lo, hc_stream), :],
                sem2.at[c])

        def body(chunk, n):
            x = x_ref[...]
            y = None
            for c in range(n):
                if chunk == hc_stream:
                    w1_copy(c).wait()
                    w2_copy(c).wait()
                lo = c * chunk
                hi = lo + chunk
                h = jnp.dot(x, w1_vmem[:, lo:hi],
                            preferred_element_type=jnp.float32)
                h = jnp.maximum(h + b1_ref[:, lo:hi], 0.0)
                p = jnp.dot(h, w2_vmem[lo:hi, :],
                            preferred_element_type=jnp.float32)
                y = p if y is None else y + p
            o_ref[...] = (y + b2_ref[...]).astype(o_ref.dtype)

        @pl.when(i == 0)
        def _():
            for c in range(n_stream):
                w1_copy(c).start()
                w2_copy(c).start()
            body(hc_stream, n_stream)

        @pl.when(i != 0)
        def _():
            body(hc, n_chunks)
    return _mlp_kernel


def kernel(x, w1, b1, w2, b2, *, batch_tile=512, h_chunk=1024):
    B, D_in = x.shape
    H = w1.shape[1]
    D_out = w2.shape[1]
    dtype = x.dtype

    b1 = b1.reshape(1, H).astype(jnp.float32)
    b2 = b2.reshape(1, D_out).astype(jnp.float32)

    D_in_p = _round_up(D_in, _LANE)
    H_p = _round_up(H, _LANE)
    D_out_p = _round_up(D_out, _LANE)
    tb = min(batch_tile, _round_up(B, _SUBLANE))
    B_p = _round_up(B, tb)

    xp = jnp.pad(x, ((0, B_p - B), (0, D_in_p - D_in)))
    w1p = jnp.pad(w1, ((0, D_in_p - D_in), (0, H_p - H)))
    b1p = jnp.pad(b1, ((0, 0), (0, H_p - H)))
    w2p = jnp.pad(w2, ((0, H_p - H), (0, D_out_p - D_out)))
    b2p = jnp.pad(b2, ((0, 0), (0, D_out_p - D_out)))

    hc = min(h_chunk, H_p)
    n_chunks = -(-H_p // hc)
    if n_chunks * hc != H_p:
        hc, n_chunks = H_p, 1

    hc_stream = min(512, hc)
    n_stream = H_p // hc_stream
    if n_stream * hc_stream != H_p:
        hc_stream, n_stream = hc, n_chunks

    n_tiles = B_p // tb

    out_p = pl.pallas_call(
        _make_mlp_kernel(hc_stream, n_stream, hc, n_chunks),
        out_shape=jax.ShapeDtypeStruct((B_p, D_out_p), dtype),
        grid=(n_tiles,),
        in_specs=[
            pl.BlockSpec((tb, D_in_p), lambda i: (i, 0)),
            pl.BlockSpec((1, H_p), lambda i: (0, 0)),
            pl.BlockSpec((1, D_out_p), lambda i: (0, 0)),
            pl.BlockSpec(memory_space=pltpu.MemorySpace.HBM),
            pl.BlockSpec(memory_space=pltpu.MemorySpace.HBM),
        ],
        out_specs=pl.BlockSpec((tb, D_out_p), lambda i: (i, 0)),
        scratch_shapes=[
            pltpu.VMEM((D_in_p, H_p), jnp.float32),
            pltpu.VMEM((H_p, D_out_p), jnp.float32),
            pltpu.SemaphoreType.DMA((n_stream,)),
            pltpu.SemaphoreType.DMA((n_stream,)),
        ],
        compiler_params=pltpu.CompilerParams(
            dimension_semantics=("arbitrary",)),
    )(xp, b1p, b2p, w1p, w2p)

    if B_p != B or D_out_p != D_out:
        out_p = out_p[:B, :D_out]
    return out_p
